# Initial kernel scaffold; baseline (speedup 1.0000x reference)
import jax
import jax.numpy as jnp
from jax.experimental import pallas as pl


def kernel(x, pe):
    raise NotImplementedError("write your pallas kernel here")


# speedup vs baseline: 1.4993x; 1.4993x over previous
import jax
import jax.numpy as jnp
from jax.experimental import pallas as pl


def _add_kernel(x_ref, pe_ref, o_ref):
    o_ref[...] = x_ref[...] + pe_ref[...]


def kernel(x, pe):
    B, S, D = x.shape
    S_BLK = 512
    return pl.pallas_call(
        _add_kernel,
        grid=(S // S_BLK, B),
        in_specs=[
            pl.BlockSpec((1, S_BLK, D), lambda i, j: (j, i, 0)),
            pl.BlockSpec((S_BLK, D), lambda i, j: (i, 0)),
        ],
        out_specs=pl.BlockSpec((1, S_BLK, D), lambda i, j: (j, i, 0)),
        out_shape=jax.ShapeDtypeStruct(x.shape, x.dtype),
    )(x, pe)


# speedup vs baseline: 1.6664x; 1.1115x over previous
import jax
import jax.numpy as jnp
from jax.experimental import pallas as pl


def _add_kernel(x_ref, pe_ref, o_ref):
    o_ref[...] = x_ref[...] + pe_ref[...]


def kernel(x, pe):
    B, S, D = x.shape
    S_BLK = 1024
    return pl.pallas_call(
        _add_kernel,
        grid=(S // S_BLK, B),
        in_specs=[
            pl.BlockSpec((1, S_BLK, D), lambda i, j: (j, i, 0)),
            pl.BlockSpec((S_BLK, D), lambda i, j: (i, 0)),
        ],
        out_specs=pl.BlockSpec((1, S_BLK, D), lambda i, j: (j, i, 0)),
        out_shape=jax.ShapeDtypeStruct(x.shape, x.dtype),
    )(x, pe)


# speedup vs baseline: 1.7396x; 1.0439x over previous
import jax
import jax.numpy as jnp
from jax.experimental import pallas as pl


def _add_kernel(x_ref, pe_ref, o_ref):
    o_ref[...] = x_ref[...] + pe_ref[...]


def kernel(x, pe):
    B, S, D = x.shape
    S_BLK = 2048
    return pl.pallas_call(
        _add_kernel,
        grid=(S // S_BLK, B),
        in_specs=[
            pl.BlockSpec((1, S_BLK, D), lambda i, j: (j, i, 0)),
            pl.BlockSpec((S_BLK, D), lambda i, j: (i, 0)),
        ],
        out_specs=pl.BlockSpec((1, S_BLK, D), lambda i, j: (j, i, 0)),
        out_shape=jax.ShapeDtypeStruct(x.shape, x.dtype),
    )(x, pe)
